# Initial kernel scaffold; baseline (speedup 1.0000x reference)
import jax
import jax.numpy as jnp
from jax.experimental import pallas as pl


def kernel(z, embedding):
    raise NotImplementedError("write your pallas kernel here")


# speedup vs baseline: 1.3347x; 1.3347x over previous
import functools

import jax
import jax.numpy as jnp
from jax.experimental import pallas as pl

_N_EMB = 1024
_DIM = 64
_BLK = 512


def _vq_body(zsq_ref, z_ref, emt_ref, esq_ref, emb_ref, ids_ref, q_ref):
    m = jnp.dot(z_ref[...], emt_ref[...], preferred_element_type=jnp.float32)
    d = (zsq_ref[...] - 2.0 * m) + esq_ref[...]
    iota = jax.lax.broadcasted_iota(jnp.int32, d.shape, 1)
    mn = jnp.min(d, axis=1, keepdims=True)
    cand = jnp.where(d == mn, iota, jnp.int32(_N_EMB))
    idx = jnp.min(cand, axis=1, keepdims=True)
    ids_ref[...] = idx
    oh = (iota == idx).astype(jnp.float32)
    q_ref[...] = jnp.dot(oh, emb_ref[...], preferred_element_type=jnp.float32)


@functools.partial(jax.jit, static_argnames=())
def kernel(z, embedding):
    bsz, seq_len, dim = z.shape
    n = bsz * seq_len
    zf = z.reshape(n, dim)
    z_sq = jnp.sum(zf * zf, axis=1, keepdims=True)
    e_sq = jnp.sum(embedding * embedding, axis=1)[None, :]
    emb_t = embedding.T

    grid = (n // _BLK,)
    ids, q = pl.pallas_call(
        _vq_body,
        grid=grid,
        in_specs=[
            pl.BlockSpec((_BLK, 1), lambda i: (i, 0)),
            pl.BlockSpec((_BLK, dim), lambda i: (i, 0)),
            pl.BlockSpec((dim, _N_EMB), lambda i: (0, 0)),
            pl.BlockSpec((1, _N_EMB), lambda i: (0, 0)),
            pl.BlockSpec((_N_EMB, dim), lambda i: (0, 0)),
        ],
        out_specs=[
            pl.BlockSpec((_BLK, 1), lambda i: (i, 0)),
            pl.BlockSpec((_BLK, dim), lambda i: (i, 0)),
        ],
        out_shape=[
            jax.ShapeDtypeStruct((n, 1), jnp.int32),
            jax.ShapeDtypeStruct((n, dim), jnp.float32),
        ],
    )(z_sq, zf, emb_t, e_sq, embedding)

    quantized = q.reshape(bsz, seq_len, dim)
    token_ids = ids.reshape(bsz, seq_len)
    return quantized, token_ids
